# baseline (speedup 1.0000x reference)
import functools

import jax
import jax.numpy as jnp
from jax import lax
from jax.experimental import pallas as pl
from jax.experimental.pallas import tpu as pltpu
from jax.experimental.pallas import tpu_sc as plsc

N = 10000
E = 320000
D_IN = 512
H1 = 800
H2 = 256

NPAD = 10240
HALF = NPAD // 2
F = 128
NCHUNK = 4
ACC_ROWS = 5248
LTRASH = HALF

C_BLOCKS = 79
C_PER_TILE = C_BLOCKS * 128
CAP_BLOCKS = 80

DEG_ACC = 20480
DEG_W = 16

GTRASH = NCHUNK * N


@functools.cache
def _make_prep_kernel():
    mesh = plsc.VectorSubcoreMesh(core_axis_name="c", subcore_axis_name="s")
    return functools.partial(
        pl.kernel,
        out_type=(
            jax.ShapeDtypeStruct((2, DEG_ACC, DEG_W), jnp.float32),
            jax.ShapeDtypeStruct((64, CAP_BLOCKS, 128), jnp.int32),
            jax.ShapeDtypeStruct((64, CAP_BLOCKS, 128), jnp.int32),
            jax.ShapeDtypeStruct((32, 16), jnp.int32),
        ),
        mesh=mesh,
        scratch_types=[
            pltpu.VMEM((C_BLOCKS, 128), jnp.int32),
            pltpu.VMEM((C_BLOCKS, 128), jnp.int32),
            pltpu.VMEM((C_BLOCKS, 128), jnp.int32),
            pltpu.VMEM((128, DEG_W), jnp.float32),
            pltpu.VMEM((128, DEG_W), jnp.float32),
            pltpu.VMEM((CAP_BLOCKS, 128), jnp.int32),
            pltpu.VMEM((CAP_BLOCKS, 128), jnp.int32),
            pltpu.VMEM((CAP_BLOCKS, 128), jnp.int32),
            pltpu.VMEM((CAP_BLOCKS, 128), jnp.int32),
            pltpu.VMEM((16,), jnp.int32),
            pltpu.VMEM_SHARED((DEG_ACC, DEG_W), jnp.float32),
        ],
        compiler_params=pltpu.CompilerParams(use_tc_tiling_on_sc=False,
                                             needs_layout_passes=False),
    )(_prep_body)


def _prep_body(src_hbm, dst_hbm, deg_hbm, srcc_hbm, dstc_hbm, cnts_hbm,
               src_v, dst_v, dsto_v, ones_v, zb_v,
               ssl, sdl, ssh, sdh, cnt_v, acc):
    c = lax.axis_index("c")
    s = lax.axis_index("s")
    w = c * 16 + s

    one = jnp.ones((DEG_W,), jnp.float32)
    zero = jnp.zeros((DEG_W,), jnp.float32)
    iota = lax.iota(jnp.int32, 16)
    strash = jnp.full((16,), GTRASH, jnp.int32)
    ltrash = jnp.full((16,), LTRASH, jnp.int32)

    def fill(i, _):
        ones_v[i, :] = one
        zb_v[i, :] = zero
        return 0

    lax.fori_loop(0, 128, fill, 0)

    def fill_stage(i, _):
        for k in range(8):
            sl = pl.ds(k * 16, 16)
            ssl[i, sl] = strash
            sdl[i, sl] = ltrash
            ssh[i, sl] = strash
            sdh[i, sl] = ltrash
        return 0

    lax.fori_loop(0, CAP_BLOCKS, fill_stage, 0)

    for p in range(10):
        pltpu.sync_copy(zb_v, acc.at[pl.ds(s * 1280 + p * 128, 128)])
    plsc.subcore_barrier()

    pltpu.sync_copy(src_hbm.at[w], src_v)
    pltpu.sync_copy(dst_hbm.at[w], dst_v)

    def mkoff(j, _):
        for k in range(8):
            sl = pl.ds(k * 16, 16)
            dsto_v[j, sl] = dst_v[j, sl] + NPAD
        return 0

    lax.fori_loop(0, C_BLOCKS, mkoff, 0)

    def dego(j, _):
        pltpu.sync_copy(ones_v, acc.at[src_v.at[j]], add=True)
        return 0

    def degi(j, _):
        pltpu.sync_copy(ones_v, acc.at[dsto_v.at[j]], add=True)
        return 0

    lax.fori_loop(0, C_BLOCKS, dego, 0)
    lax.fori_loop(0, C_BLOCKS, degi, 0)

    zvec = jnp.zeros((16,), jnp.int32)

    def cbody(j, carry):
        off_l, off_h = carry
        for k in range(8):
            sl = pl.ds(k * 16, 16)
            sv = src_v[j, sl] * NCHUNK
            dv = dst_v[j, sl]
            m = dv < HALF
            n_l = plsc.all_reduce_population_count(m)
            cum_l = plsc.cumsum(m.astype(jnp.int32))
            cum_h = (iota + 1) - cum_l
            idx_l = off_l + cum_l - 1
            idx_h = off_h + cum_h - 1
            plsc.store_scatter(ssl, [idx_l >> 7, idx_l & 127], sv, mask=m)
            plsc.store_scatter(sdl, [idx_l >> 7, idx_l & 127], dv, mask=m)
            nm = jnp.logical_not(m)
            plsc.store_scatter(ssh, [idx_h >> 7, idx_h & 127], sv, mask=nm)
            plsc.store_scatter(sdh, [idx_h >> 7, idx_h & 127], dv - HALF,
                               mask=nm)
            off_l = off_l + n_l
            off_h = off_h + (16 - n_l)
        return off_l, off_h

    off_l, off_h = lax.fori_loop(0, C_BLOCKS, cbody, (zvec, zvec))

    cnt_v[...] = jnp.where(iota == 0, off_l,
                           jnp.where(iota == 1, off_h, 0))
    pltpu.sync_copy(cnt_v, cnts_hbm.at[w])
    pltpu.sync_copy(ssl, srcc_hbm.at[w])
    pltpu.sync_copy(sdl, dstc_hbm.at[w])
    pltpu.sync_copy(ssh, srcc_hbm.at[32 + w])
    pltpu.sync_copy(sdh, dstc_hbm.at[32 + w])

    plsc.subcore_barrier()
    for p in range(10):
        pltpu.sync_copy(acc.at[pl.ds(s * 1280 + p * 128, 128)], zb_v)
        pltpu.sync_copy(zb_v, deg_hbm.at[c, pl.ds(s * 1280 + p * 128, 128)])


@functools.cache
def _make_scatter_kernel():
    mesh = plsc.VectorSubcoreMesh(core_axis_name="c", subcore_axis_name="s")
    return functools.partial(
        pl.kernel,
        out_type=jax.ShapeDtypeStruct((NPAD, NCHUNK, F), jnp.float32),
        mesh=mesh,
        scratch_types=[
            pltpu.VMEM((CAP_BLOCKS, 128), jnp.int32),
            pltpu.VMEM((CAP_BLOCKS, 128), jnp.int32),
            pltpu.VMEM((CAP_BLOCKS, 128), jnp.int32),
            pltpu.VMEM((CAP_BLOCKS, 128), jnp.int32),
            pltpu.VMEM((16,), jnp.int32),
            pltpu.VMEM((16,), jnp.int32),
            pltpu.VMEM((128, F), jnp.float32),
            pltpu.VMEM((128, F), jnp.float32),
            pltpu.VMEM((64, F), jnp.float32),
            pltpu.VMEM_SHARED((ACC_ROWS, F), jnp.float32),
            pltpu.SemaphoreType.DMA,
            pltpu.SemaphoreType.DMA,
        ],
        compiler_params=pltpu.CompilerParams(use_tc_tiling_on_sc=False,
                                             needs_layout_passes=False),
    )(_scatter_body)


def _scatter_body(table_hbm, srcc_hbm, dstc_hbm, cnts_hbm, out_hbm,
                  sa_v, da_v, sb_v, db_v, ca_v, cb_v,
                  buf_a, buf_b, zb_v, acc, sem_a, sem_b):
    c = lax.axis_index("c")
    s = lax.axis_index("s")
    iota = lax.iota(jnp.int32, 16)

    zero = jnp.zeros((16,), jnp.float32)

    def fill(i, _):
        for k in range(F // 16):
            zb_v[i, pl.ds(k * 16, 16)] = zero
        return 0

    lax.fori_loop(0, 64, fill, 0)

    r0 = 2 * s
    r1 = 2 * s + 1
    pltpu.sync_copy(srcc_hbm.at[c * 32 + r0], sa_v)
    pltpu.sync_copy(dstc_hbm.at[c * 32 + r0], da_v)
    pltpu.sync_copy(srcc_hbm.at[c * 32 + r1], sb_v)
    pltpu.sync_copy(dstc_hbm.at[c * 32 + r1], db_v)
    pltpu.sync_copy(cnts_hbm.at[r0], ca_v)
    pltpu.sync_copy(cnts_hbm.at[r1], cb_v)

    cnt_a = jnp.max(jnp.where(iota == c, ca_v[...], 0))
    cnt_b = jnp.max(jnp.where(iota == c, cb_v[...], 0))
    nb_a = (cnt_a + 127) >> 7
    nb_b = (cnt_b + 127) >> 7

    def run_region(sv, dv, nb):
        @pl.when(nb >= 1)
        def _():
            pltpu.async_copy(table_hbm.at[sv.at[0]], buf_a, sem_a)

        @pl.when(nb >= 2)
        def _():
            pltpu.async_copy(table_hbm.at[sv.at[1]], buf_b, sem_b)

        def body(i, _):
            j0 = 2 * i

            @pl.when(j0 < nb)
            def _():
                pltpu.make_async_copy(
                    table_hbm.at[sv.at[j0]], buf_a, sem_a).wait()
                pltpu.sync_copy(buf_a, acc.at[dv.at[j0]], add=True)

                @pl.when(j0 + 2 < nb)
                def _():
                    pltpu.async_copy(table_hbm.at[sv.at[j0 + 2]], buf_a, sem_a)

            @pl.when(j0 + 1 < nb)
            def _():
                pltpu.make_async_copy(
                    table_hbm.at[sv.at[j0 + 1]], buf_b, sem_b).wait()
                pltpu.sync_copy(buf_b, acc.at[dv.at[j0 + 1]], add=True)

                @pl.when(j0 + 3 < nb)
                def _():
                    pltpu.async_copy(table_hbm.at[sv.at[j0 + 3]], buf_b, sem_b)

            return 0

        lax.fori_loop(0, CAP_BLOCKS // 2, body, 0)

    def bump(sv):
        def mk(j, _):
            for k in range(8):
                sl = pl.ds(k * 16, 16)
                sv[j, sl] = sv[j, sl] + 1
            return 0

        lax.fori_loop(0, CAP_BLOCKS, mk, 0)

    for chunk in range(NCHUNK):
        if chunk > 0:
            bump(sa_v)
            bump(sb_v)
        for p in range(5):
            pltpu.sync_copy(zb_v, acc.at[pl.ds(s * 328 + p * 64, 64)])
        pltpu.sync_copy(zb_v.at[pl.ds(0, 8)], acc.at[pl.ds(s * 328 + 320, 8)])
        plsc.subcore_barrier()

        run_region(sa_v, da_v, nb_a)
        run_region(sb_v, db_v, nb_b)
        plsc.subcore_barrier()

        base = s * 320
        gbase = c * HALF + base
        for (off, rows) in ((0, 128), (128, 128), (256, 64)):
            pltpu.sync_copy(acc.at[pl.ds(base + off, rows)],
                            buf_a.at[pl.ds(0, rows)])
            pltpu.sync_copy(buf_a.at[pl.ds(0, rows)],
                            out_hbm.at[pl.ds(gbase + off, rows), chunk])
        plsc.subcore_barrier()


_HI = lax.Precision.HIGHEST


def _prep_body_tc(f_ref, dout_ref, o_ref):
    scale = lax.rsqrt(jnp.maximum(dout_ref[...], 1.0))
    o_ref[...] = f_ref[...] * scale


def _mid_body(s1_ref, din_ref, dout_ref, w1_ref, b1_ref, w23_ref, o_ref):
    din = lax.rsqrt(jnp.maximum(din_ref[...], 1.0))
    dout = lax.rsqrt(jnp.maximum(dout_ref[...], 1.0))
    a = s1_ref[...] * din
    h = jnp.maximum(jnp.dot(a, w1_ref[...], precision=_HI) + b1_ref[...], 0.0)
    o_ref[...] = jnp.dot(h * dout, w23_ref[...], precision=_HI)


def _z_body(s2_ref, din_ref, noise_ref, b2_ref, b3_ref, zo_ref):
    din = lax.rsqrt(jnp.maximum(din_ref[...], 1.0))
    s2n = s2_ref[...] * din
    mean = s2n[:, :H2] + b2_ref[...]
    logs = s2n[:, H2:] + b3_ref[...]
    zo_ref[...] = mean + noise_ref[...] * jnp.exp(logs)


def _dec_body(zi_ref, zj_ref, o_ref):
    a = lax.dot_general(zi_ref[...], zj_ref[...], (((1,), (1,)), ((), ())),
                        precision=lax.Precision.DEFAULT,
                        preferred_element_type=jnp.float32)
    o_ref[...] = jax.nn.sigmoid(a)


@jax.jit
def kernel(features, edge_index, noise, W1, b1, W2, b2, W3, b3):
    src = edge_index[0]
    dst = edge_index[1]

    npad_e = 32 * C_PER_TILE - E
    padrows = N + jnp.arange(npad_e, dtype=jnp.int32) % (NPAD - N)
    src_slabs = jnp.concatenate([src, padrows]).reshape(32, C_BLOCKS, 128)
    dst_slabs = jnp.concatenate([dst, padrows]).reshape(32, C_BLOCKS, 128)

    degp, srcc, dstc, cnts = _make_prep_kernel()(src_slabs, dst_slabs)
    dsum = degp[0, :, 0] + degp[1, :, 0]
    deg_out = jnp.pad(dsum[:N], (0, NPAD - N), constant_values=1.0)
    deg_out = deg_out.reshape(NPAD, 1)
    deg_in = jnp.pad(dsum[NPAD:NPAD + N], (0, NPAD - N), constant_values=1.0)
    deg_in = deg_in.reshape(NPAD, 1)

    feats_pad = jnp.pad(features, ((0, NPAD - N), (0, 0)))
    xn = pl.pallas_call(
        _prep_body_tc,
        grid=(NPAD // 512,),
        in_specs=[
            pl.BlockSpec((512, D_IN), lambda i: (i, 0)),
            pl.BlockSpec((512, 1), lambda i: (i, 0)),
        ],
        out_specs=pl.BlockSpec((512, D_IN), lambda i: (i, 0)),
        out_shape=jax.ShapeDtypeStruct((NPAD, D_IN), jnp.float32),
    )(feats_pad, deg_out)

    s1 = _make_scatter_kernel()(
        xn.reshape(NCHUNK * NPAD, F), srcc, dstc, cnts)
    s1 = s1.reshape(NPAD, D_IN)

    w23 = jnp.concatenate([W2, W3], axis=1)
    t = pl.pallas_call(
        _mid_body,
        grid=(NPAD // 512,),
        in_specs=[
            pl.BlockSpec((512, D_IN), lambda i: (i, 0)),
            pl.BlockSpec((512, 1), lambda i: (i, 0)),
            pl.BlockSpec((512, 1), lambda i: (i, 0)),
            pl.BlockSpec((D_IN, H1), lambda i: (0, 0)),
            pl.BlockSpec((1, H1), lambda i: (0, 0)),
            pl.BlockSpec((H1, 2 * H2), lambda i: (0, 0)),
        ],
        out_specs=pl.BlockSpec((512, D_IN), lambda i: (i, 0)),
        out_shape=jax.ShapeDtypeStruct((NPAD, D_IN), jnp.float32),
    )(s1, deg_in, deg_out, W1, b1.reshape(1, H1), w23)

    s2 = _make_scatter_kernel()(
        t.reshape(NCHUNK * NPAD, F), srcc, dstc, cnts)
    s2 = s2.reshape(NPAD, D_IN)

    noise_pad = jnp.pad(noise, ((0, NPAD - N), (0, 0)))
    zp = pl.pallas_call(
        _z_body,
        grid=(NPAD // 512,),
        in_specs=[
            pl.BlockSpec((512, D_IN), lambda i: (i, 0)),
            pl.BlockSpec((512, 1), lambda i: (i, 0)),
            pl.BlockSpec((512, H2), lambda i: (i, 0)),
            pl.BlockSpec((1, H2), lambda i: (0, 0)),
            pl.BlockSpec((1, H2), lambda i: (0, 0)),
        ],
        out_specs=pl.BlockSpec((512, H2), lambda i: (i, 0)),
        out_shape=jax.ShapeDtypeStruct((NPAD, H2), jnp.float32),
    )(s2, deg_in, noise_pad, b2.reshape(1, H2), b3.reshape(1, H2))

    adj = pl.pallas_call(
        _dec_body,
        grid=(pl.cdiv(N, 512), pl.cdiv(N, 1024)),
        in_specs=[
            pl.BlockSpec((512, H2), lambda i, j: (i, 0)),
            pl.BlockSpec((1024, H2), lambda i, j: (j, 0)),
        ],
        out_specs=pl.BlockSpec((512, 1024), lambda i, j: (i, j)),
        out_shape=jax.ShapeDtypeStruct((N, N), jnp.float32),
    )(zp, zp)

    return zp[:N], adj


# speedup vs baseline: 1.1263x; 1.1263x over previous
import functools

import jax
import jax.numpy as jnp
from jax import lax
from jax.experimental import pallas as pl
from jax.experimental.pallas import tpu as pltpu
from jax.experimental.pallas import tpu_sc as plsc

N = 10000
E = 320000
D_IN = 512
H1 = 800
H2 = 256

NPAD = 10240
TRASH = N
F1 = 64
F2 = 64

DEG_ACC = 20480
DEG_TRASH = 2 * N
DEG_BLOCKS = 157
DEG_PER_TILE = DEG_BLOCKS * 128
DEG_W = 16

SC_BLOCKS = 158
SC_PER_TILE = SC_BLOCKS * 128

@functools.cache
def _make_deg_kernel():
    mesh = plsc.VectorSubcoreMesh(core_axis_name="c", subcore_axis_name="s")
    return functools.partial(
        pl.kernel,
        out_type=jax.ShapeDtypeStruct((2, DEG_ACC, DEG_W), jnp.float32),
        mesh=mesh,
        scratch_types=[
            pltpu.VMEM((DEG_BLOCKS, 128), jnp.int32),
            pltpu.VMEM((128, DEG_W), jnp.float32),
            pltpu.VMEM((128, DEG_W), jnp.float32),
            pltpu.VMEM_SHARED((DEG_ACC, DEG_W), jnp.float32),
        ],
        compiler_params=pltpu.CompilerParams(use_tc_tiling_on_sc=False),
    )(_deg_body)


def _deg_body(idx_hbm, out_hbm, idx_v, ones_v, zb_v, acc):
    c = lax.axis_index("c")
    s = lax.axis_index("s")
    w = c * 16 + s

    one = jnp.ones((DEG_W,), jnp.float32)
    zero = jnp.zeros((DEG_W,), jnp.float32)

    def fill(i, _):
        ones_v[i, :] = one
        zb_v[i, :] = zero
        return 0

    lax.fori_loop(0, 128, fill, 0)

    for p in range(10):
        pltpu.sync_copy(zb_v, acc.at[pl.ds(s * 1280 + p * 128, 128)])
    plsc.subcore_barrier()

    pltpu.sync_copy(idx_hbm.at[w], idx_v)

    def body(j, _):
        pltpu.sync_copy(ones_v, acc.at[idx_v.at[j]], add=True)
        return 0

    lax.fori_loop(0, DEG_BLOCKS, body, 0)
    plsc.subcore_barrier()

    for p in range(10):
        pltpu.sync_copy(acc.at[pl.ds(s * 1280 + p * 128, 128)], zb_v)
        pltpu.sync_copy(zb_v, out_hbm.at[c, pl.ds(s * 1280 + p * 128, 128)])


@functools.cache
def _make_scatter_kernel(f):
    nchunk = D_IN // f
    mesh = plsc.VectorSubcoreMesh(core_axis_name="c", subcore_axis_name="s")
    return functools.partial(
        pl.kernel,
        out_type=jax.ShapeDtypeStruct((NPAD, nchunk, f), jnp.float32),
        mesh=mesh,
        scratch_types=[
            pltpu.VMEM((SC_BLOCKS, 128), jnp.int32),
            pltpu.VMEM((SC_BLOCKS, 128), jnp.int32),
            pltpu.VMEM((128, f), jnp.float32),
            pltpu.VMEM((128, f), jnp.float32),
            pltpu.VMEM((64, f), jnp.float32),
            pltpu.VMEM_SHARED((NPAD, f), jnp.float32),
            pltpu.SemaphoreType.DMA,
            pltpu.SemaphoreType.DMA,
        ],
        compiler_params=pltpu.CompilerParams(use_tc_tiling_on_sc=False),
    )(functools.partial(_scatter_body, f))


def _scatter_body(f, table_hbm, srcs4_hbm, dsts_hbm, out_hbm,
                  src_v, dst_v, buf_a, buf_b, zb_v, acc, sem_a, sem_b):
    nchunk = D_IN // f
    cpc = nchunk // 2
    c = lax.axis_index("c")
    s = lax.axis_index("s")

    zero = jnp.zeros((16,), jnp.float32)

    def fill(i, _):
        for k in range(f // 16):
            zb_v[i, pl.ds(k * 16, 16)] = zero
        return 0

    lax.fori_loop(0, 64, fill, 0)

    pltpu.sync_copy(dsts_hbm.at[s], dst_v)

    for lc in range(cpc):
        chunk = cpc * c + lc
        for p in range(10):
            pltpu.sync_copy(zb_v, acc.at[pl.ds(s * 640 + p * 64, 64)])
        pltpu.sync_copy(srcs4_hbm.at[chunk * 16 + s], src_v)
        plsc.subcore_barrier()

        pltpu.async_copy(table_hbm.at[src_v.at[0]], buf_a, sem_a)
        pltpu.async_copy(table_hbm.at[src_v.at[1]], buf_b, sem_b)

        def body(i, _):
            j0 = 2 * i
            pltpu.make_async_copy(
                table_hbm.at[src_v.at[j0]], buf_a, sem_a).wait()
            pltpu.sync_copy(buf_a, acc.at[dst_v.at[j0]], add=True)

            @pl.when(j0 + 2 < SC_BLOCKS)
            def _():
                pltpu.async_copy(table_hbm.at[src_v.at[j0 + 2]], buf_a, sem_a)

            pltpu.make_async_copy(
                table_hbm.at[src_v.at[j0 + 1]], buf_b, sem_b).wait()
            pltpu.sync_copy(buf_b, acc.at[dst_v.at[j0 + 1]], add=True)

            @pl.when(j0 + 3 < SC_BLOCKS)
            def _():
                pltpu.async_copy(table_hbm.at[src_v.at[j0 + 3]], buf_b, sem_b)

            return 0

        lax.fori_loop(0, SC_BLOCKS // 2, body, 0)
        plsc.subcore_barrier()

        for p in range(5):
            pltpu.sync_copy(acc.at[pl.ds(s * 640 + p * 128, 128)], buf_a)
            pltpu.sync_copy(
                buf_a, out_hbm.at[pl.ds(s * 640 + p * 128, 128), chunk])
        plsc.subcore_barrier()


_HI = lax.Precision.HIGHEST


def _prep_body(f_ref, dout_ref, o_ref):
    scale = lax.rsqrt(jnp.maximum(dout_ref[...], 1.0))
    o_ref[...] = f_ref[...] * scale


def _mid_body(s1_ref, din_ref, dout_ref, w1_ref, b1_ref, w23_ref, o_ref):
    din = lax.rsqrt(jnp.maximum(din_ref[...], 1.0))
    dout = lax.rsqrt(jnp.maximum(dout_ref[...], 1.0))
    a = s1_ref[...] * din
    h = jnp.maximum(jnp.dot(a, w1_ref[...], precision=_HI) + b1_ref[...], 0.0)
    o_ref[...] = jnp.dot(h * dout, w23_ref[...], precision=_HI)


def _z_body(s2_ref, din_ref, noise_ref, b2_ref, b3_ref, zo_ref):
    din = lax.rsqrt(jnp.maximum(din_ref[...], 1.0))
    s2n = s2_ref[...] * din
    mean = s2n[:, :H2] + b2_ref[...]
    logs = s2n[:, H2:] + b3_ref[...]
    zo_ref[...] = mean + noise_ref[...] * jnp.exp(logs)


def _dec_body(zi_ref, zj_ref, o_ref):
    a = lax.dot_general(zi_ref[...], zj_ref[...], (((1,), (1,)), ((), ())),
                        precision=lax.Precision.DEFAULT,
                        preferred_element_type=jnp.float32)
    o_ref[...] = jax.nn.sigmoid(a)


@jax.jit
def kernel(features, edge_index, noise, W1, b1, W2, b2, W3, b3):
    src = edge_index[0]
    dst = edge_index[1]

    cat = jnp.concatenate([src, dst + N])
    npad_d = 32 * DEG_PER_TILE - 2 * E
    cat = jnp.concatenate(
        [cat, DEG_TRASH + jnp.arange(npad_d, dtype=jnp.int32)
              % (DEG_ACC - DEG_TRASH)])
    deg_idx = cat.reshape(32, DEG_BLOCKS, 128)

    degp = _make_deg_kernel()(deg_idx)
    dsum = degp[0, :, 0] + degp[1, :, 0]
    deg_out = jnp.pad(dsum[:N], (0, NPAD - N), constant_values=1.0)
    deg_out = deg_out.reshape(NPAD, 1)
    deg_in = jnp.pad(dsum[N:2 * N], (0, NPAD - N), constant_values=1.0)
    deg_in = deg_in.reshape(NPAD, 1)

    src_pad = jnp.concatenate(
        [src, jnp.zeros((16 * SC_PER_TILE - E,), jnp.int32)])
    npad_e = 16 * SC_PER_TILE - E
    trash_rows = TRASH + jnp.arange(npad_e, dtype=jnp.int32) % (NPAD - N)
    dst_pad = jnp.concatenate([dst, trash_rows])
    src_slab = src_pad.reshape(16, SC_BLOCKS, 128)

    def chunked_srcs(nchunk):
        a = (src_slab[None] * nchunk +
             jnp.arange(nchunk, dtype=jnp.int32)[:, None, None, None])
        return a.reshape(nchunk * 16, SC_BLOCKS, 128)

    srcs4 = chunked_srcs(D_IN // F1)
    srcs8 = chunked_srcs(D_IN // F2)
    dsts = dst_pad.reshape(16, SC_BLOCKS, 128)

    feats_pad = jnp.pad(features, ((0, NPAD - N), (0, 0)))
    xn = pl.pallas_call(
        _prep_body,
        grid=(NPAD // 512,),
        in_specs=[
            pl.BlockSpec((512, D_IN), lambda i: (i, 0)),
            pl.BlockSpec((512, 1), lambda i: (i, 0)),
        ],
        out_specs=pl.BlockSpec((512, D_IN), lambda i: (i, 0)),
        out_shape=jax.ShapeDtypeStruct((NPAD, D_IN), jnp.float32),
    )(feats_pad, deg_out)

    s1 = _make_scatter_kernel(F1)(
        xn.reshape((D_IN // F1) * NPAD, F1), srcs4, dsts)
    s1 = s1.reshape(NPAD, D_IN)

    w23 = jnp.concatenate([W2, W3], axis=1)
    t = pl.pallas_call(
        _mid_body,
        grid=(NPAD // 512,),
        in_specs=[
            pl.BlockSpec((512, D_IN), lambda i: (i, 0)),
            pl.BlockSpec((512, 1), lambda i: (i, 0)),
            pl.BlockSpec((512, 1), lambda i: (i, 0)),
            pl.BlockSpec((D_IN, H1), lambda i: (0, 0)),
            pl.BlockSpec((1, H1), lambda i: (0, 0)),
            pl.BlockSpec((H1, 2 * H2), lambda i: (0, 0)),
        ],
        out_specs=pl.BlockSpec((512, D_IN), lambda i: (i, 0)),
        out_shape=jax.ShapeDtypeStruct((NPAD, D_IN), jnp.float32),
    )(s1, deg_in, deg_out, W1, b1.reshape(1, H1), w23)

    s2 = _make_scatter_kernel(F2)(
        t.reshape((D_IN // F2) * NPAD, F2), srcs8, dsts)
    s2 = s2.reshape(NPAD, D_IN)

    noise_pad = jnp.pad(noise, ((0, NPAD - N), (0, 0)))
    zp = pl.pallas_call(
        _z_body,
        grid=(NPAD // 512,),
        in_specs=[
            pl.BlockSpec((512, D_IN), lambda i: (i, 0)),
            pl.BlockSpec((512, 1), lambda i: (i, 0)),
            pl.BlockSpec((512, H2), lambda i: (i, 0)),
            pl.BlockSpec((1, H2), lambda i: (0, 0)),
            pl.BlockSpec((1, H2), lambda i: (0, 0)),
        ],
        out_specs=pl.BlockSpec((512, H2), lambda i: (i, 0)),
        out_shape=jax.ShapeDtypeStruct((NPAD, H2), jnp.float32),
    )(s2, deg_in, noise_pad, b2.reshape(1, H2), b3.reshape(1, H2))

    adj = pl.pallas_call(
        _dec_body,
        grid=(pl.cdiv(N, 512), pl.cdiv(N, 1024)),
        in_specs=[
            pl.BlockSpec((512, H2), lambda i, j: (i, 0)),
            pl.BlockSpec((1024, H2), lambda i, j: (j, 0)),
        ],
        out_specs=pl.BlockSpec((512, 1024), lambda i, j: (i, j)),
        out_shape=jax.ShapeDtypeStruct((N, N), jnp.float32),
    )(zp, zp)

    return zp[:N], adj


# speedup vs baseline: 1.1508x; 1.0218x over previous
import functools

import jax
import jax.numpy as jnp
from jax import lax
from jax.experimental import pallas as pl
from jax.experimental.pallas import tpu as pltpu
from jax.experimental.pallas import tpu_sc as plsc

N = 10000
E = 320000
D_IN = 512
H1 = 800
H2 = 256

NPAD = 10240
TRASH = N
F1 = 64
F2 = 64

DEG_ACC = 20480
DEG_TRASH = 2 * N
DEG_BLOCKS = 157
DEG_PER_TILE = DEG_BLOCKS * 128
DEG_W = 16

SC_BLOCKS = 158
SC_PER_TILE = SC_BLOCKS * 128

@functools.cache
def _make_deg_kernel():
    mesh = plsc.VectorSubcoreMesh(core_axis_name="c", subcore_axis_name="s")
    return functools.partial(
        pl.kernel,
        out_type=jax.ShapeDtypeStruct((2, DEG_ACC, DEG_W), jnp.float32),
        mesh=mesh,
        scratch_types=[
            pltpu.VMEM((DEG_BLOCKS, 128), jnp.int32),
            pltpu.VMEM((128, DEG_W), jnp.float32),
            pltpu.VMEM((128, DEG_W), jnp.float32),
            pltpu.VMEM_SHARED((DEG_ACC, DEG_W), jnp.float32),
        ],
        compiler_params=pltpu.CompilerParams(use_tc_tiling_on_sc=False),
    )(_deg_body)


def _deg_body(idx_hbm, out_hbm, idx_v, ones_v, zb_v, acc):
    c = lax.axis_index("c")
    s = lax.axis_index("s")
    w = c * 16 + s

    one = jnp.ones((DEG_W,), jnp.float32)
    zero = jnp.zeros((DEG_W,), jnp.float32)

    def fill(i, _):
        ones_v[i, :] = one
        zb_v[i, :] = zero
        return 0

    lax.fori_loop(0, 128, fill, 0)

    for p in range(10):
        pltpu.sync_copy(zb_v, acc.at[pl.ds(s * 1280 + p * 128, 128)])
    plsc.subcore_barrier()

    pltpu.sync_copy(idx_hbm.at[w], idx_v)

    def body(j, _):
        pltpu.sync_copy(ones_v, acc.at[idx_v.at[j]], add=True)
        return 0

    lax.fori_loop(0, DEG_BLOCKS, body, 0)
    plsc.subcore_barrier()

    for p in range(10):
        pltpu.sync_copy(acc.at[pl.ds(s * 1280 + p * 128, 128)], zb_v)
        pltpu.sync_copy(zb_v, out_hbm.at[c, pl.ds(s * 1280 + p * 128, 128)])


@functools.cache
def _make_scatter_kernel(f):
    nchunk = D_IN // f
    mesh = plsc.VectorSubcoreMesh(core_axis_name="c", subcore_axis_name="s")
    return functools.partial(
        pl.kernel,
        out_type=jax.ShapeDtypeStruct((NPAD, nchunk, f), jnp.float32),
        mesh=mesh,
        scratch_types=[
            pltpu.VMEM((SC_BLOCKS, 128), jnp.int32),
            pltpu.VMEM((SC_BLOCKS, 128), jnp.int32),
            pltpu.VMEM((128, f), jnp.float32),
            pltpu.VMEM((128, f), jnp.float32),
            pltpu.VMEM((64, f), jnp.float32),
            pltpu.VMEM_SHARED((NPAD, f), jnp.float32),
            pltpu.SemaphoreType.DMA,
            pltpu.SemaphoreType.DMA,
        ],
        compiler_params=pltpu.CompilerParams(use_tc_tiling_on_sc=False),
    )(functools.partial(_scatter_body, f))


def _scatter_body(f, table_hbm, srcs4_hbm, dsts_hbm, out_hbm,
                  src_v, dst_v, buf_a, buf_b, zb_v, acc, sem_a, sem_b):
    nchunk = D_IN // f
    cpc = nchunk // 2
    c = lax.axis_index("c")
    s = lax.axis_index("s")

    zero = jnp.zeros((16,), jnp.float32)

    def fill(i, _):
        for k in range(f // 16):
            zb_v[i, pl.ds(k * 16, 16)] = zero
        return 0

    lax.fori_loop(0, 64, fill, 0)

    pltpu.sync_copy(dsts_hbm.at[s], dst_v)

    for lc in range(cpc):
        chunk = cpc * c + lc
        for p in range(10):
            pltpu.sync_copy(zb_v, acc.at[pl.ds(s * 640 + p * 64, 64)])
        pltpu.sync_copy(srcs4_hbm.at[chunk * 16 + s], src_v)
        plsc.subcore_barrier()

        pltpu.async_copy(table_hbm.at[src_v.at[0]], buf_a, sem_a)
        pltpu.async_copy(table_hbm.at[src_v.at[1]], buf_b, sem_b)

        def body(i, _):
            j0 = 2 * i
            pltpu.make_async_copy(
                table_hbm.at[src_v.at[j0]], buf_a, sem_a).wait()
            pltpu.sync_copy(buf_a, acc.at[dst_v.at[j0]], add=True)

            @pl.when(j0 + 2 < SC_BLOCKS)
            def _():
                pltpu.async_copy(table_hbm.at[src_v.at[j0 + 2]], buf_a, sem_a)

            pltpu.make_async_copy(
                table_hbm.at[src_v.at[j0 + 1]], buf_b, sem_b).wait()
            pltpu.sync_copy(buf_b, acc.at[dst_v.at[j0 + 1]], add=True)

            @pl.when(j0 + 3 < SC_BLOCKS)
            def _():
                pltpu.async_copy(table_hbm.at[src_v.at[j0 + 3]], buf_b, sem_b)

            return 0

        lax.fori_loop(0, SC_BLOCKS // 2, body, 0)
        plsc.subcore_barrier()

        for p in range(5):
            pltpu.sync_copy(acc.at[pl.ds(s * 640 + p * 128, 128)], buf_a)
            pltpu.sync_copy(
                buf_a, out_hbm.at[pl.ds(s * 640 + p * 128, 128), chunk])
        plsc.subcore_barrier()


_HI = lax.Precision.HIGHEST


def _prep_body(f_ref, dout_ref, o_ref):
    scale = lax.rsqrt(jnp.maximum(dout_ref[...], 1.0))
    o_ref[...] = f_ref[...] * scale


def _mid_body(s1_ref, din_ref, dout_ref, w1_ref, b1_ref, w23_ref, o_ref):
    din = lax.rsqrt(jnp.maximum(din_ref[...], 1.0))
    dout = lax.rsqrt(jnp.maximum(dout_ref[...], 1.0))
    a = s1_ref[...] * din
    h = jnp.maximum(jnp.dot(a, w1_ref[...], precision=_HI) + b1_ref[...], 0.0)
    o_ref[...] = jnp.dot(h * dout, w23_ref[...], precision=_HI)


def _z_body(s2_ref, din_ref, noise_ref, b2_ref, b3_ref, zo_ref):
    din = lax.rsqrt(jnp.maximum(din_ref[...], 1.0))
    s2n = s2_ref[...] * din
    mean = s2n[:, :H2] + b2_ref[...]
    logs = s2n[:, H2:] + b3_ref[...]
    zo_ref[...] = mean + noise_ref[...] * jnp.exp(logs)


def _dec_body(zi_ref, zj_ref, o_ref):
    a = lax.dot_general(zi_ref[...], zj_ref[...], (((1,), (1,)), ((), ())),
                        precision=lax.Precision.DEFAULT,
                        preferred_element_type=jnp.float32)
    o_ref[...] = jax.nn.sigmoid(a)


@jax.jit
def kernel(features, edge_index, noise, W1, b1, W2, b2, W3, b3):
    src = edge_index[0]
    dst = edge_index[1]

    cat = jnp.concatenate([src, dst + N])
    npad_d = 32 * DEG_PER_TILE - 2 * E
    cat = jnp.concatenate(
        [cat, DEG_TRASH + jnp.arange(npad_d, dtype=jnp.int32)
              % (DEG_ACC - DEG_TRASH)])
    deg_idx = cat.reshape(32, DEG_BLOCKS, 128)

    degp = _make_deg_kernel()(deg_idx)
    dsum = degp[0, :, 0] + degp[1, :, 0]
    deg_out = jnp.pad(dsum[:N], (0, NPAD - N), constant_values=1.0)
    deg_out = deg_out.reshape(NPAD, 1)
    deg_in = jnp.pad(dsum[N:2 * N], (0, NPAD - N), constant_values=1.0)
    deg_in = deg_in.reshape(NPAD, 1)

    src_pad = jnp.concatenate(
        [src, jnp.zeros((16 * SC_PER_TILE - E,), jnp.int32)])
    npad_e = 16 * SC_PER_TILE - E
    trash_rows = TRASH + jnp.arange(npad_e, dtype=jnp.int32) % (NPAD - N)
    dst_pad = jnp.concatenate([dst, trash_rows])
    src_slab = src_pad.reshape(16, SC_BLOCKS, 128)

    def chunked_srcs(nchunk):
        a = (src_slab[None] * nchunk +
             jnp.arange(nchunk, dtype=jnp.int32)[:, None, None, None])
        return a.reshape(nchunk * 16, SC_BLOCKS, 128)

    srcs4 = chunked_srcs(D_IN // F1)
    srcs8 = chunked_srcs(D_IN // F2)
    dsts = dst_pad.reshape(16, SC_BLOCKS, 128)

    feats_pad = jnp.pad(features, ((0, NPAD - N), (0, 0)))
    xn = pl.pallas_call(
        _prep_body,
        grid=(NPAD // 512,),
        in_specs=[
            pl.BlockSpec((512, D_IN), lambda i: (i, 0)),
            pl.BlockSpec((512, 1), lambda i: (i, 0)),
        ],
        out_specs=pl.BlockSpec((512, D_IN), lambda i: (i, 0)),
        out_shape=jax.ShapeDtypeStruct((NPAD, D_IN), jnp.float32),
    )(feats_pad, deg_out)

    s1 = _make_scatter_kernel(F1)(
        xn.reshape((D_IN // F1) * NPAD, F1), srcs4, dsts)
    s1 = s1.reshape(NPAD, D_IN)

    w23 = jnp.concatenate([W2, W3], axis=1)
    t = pl.pallas_call(
        _mid_body,
        grid=(NPAD // 512,),
        in_specs=[
            pl.BlockSpec((512, D_IN), lambda i: (i, 0)),
            pl.BlockSpec((512, 1), lambda i: (i, 0)),
            pl.BlockSpec((512, 1), lambda i: (i, 0)),
            pl.BlockSpec((D_IN, H1), lambda i: (0, 0)),
            pl.BlockSpec((1, H1), lambda i: (0, 0)),
            pl.BlockSpec((H1, 2 * H2), lambda i: (0, 0)),
        ],
        out_specs=pl.BlockSpec((512, D_IN), lambda i: (i, 0)),
        out_shape=jax.ShapeDtypeStruct((NPAD, D_IN), jnp.float32),
    )(s1, deg_in, deg_out, W1, b1.reshape(1, H1), w23)

    s2 = _make_scatter_kernel(F2)(
        t.reshape((D_IN // F2) * NPAD, F2), srcs8, dsts)
    s2 = s2.reshape(NPAD, D_IN)

    noise_pad = jnp.pad(noise, ((0, NPAD - N), (0, 0)))
    zp = pl.pallas_call(
        _z_body,
        grid=(NPAD // 512,),
        in_specs=[
            pl.BlockSpec((512, D_IN), lambda i: (i, 0)),
            pl.BlockSpec((512, 1), lambda i: (i, 0)),
            pl.BlockSpec((512, H2), lambda i: (i, 0)),
            pl.BlockSpec((1, H2), lambda i: (0, 0)),
            pl.BlockSpec((1, H2), lambda i: (0, 0)),
        ],
        out_specs=pl.BlockSpec((512, H2), lambda i: (i, 0)),
        out_shape=jax.ShapeDtypeStruct((NPAD, H2), jnp.float32),
    )(s2, deg_in, noise_pad, b2.reshape(1, H2), b3.reshape(1, H2))

    adj = pl.pallas_call(
        _dec_body,
        grid=(pl.cdiv(N, 512), pl.cdiv(N, 2048)),
        in_specs=[
            pl.BlockSpec((512, H2), lambda i, j: (i, 0)),
            pl.BlockSpec((2048, H2), lambda i, j: (j, 0)),
        ],
        out_specs=pl.BlockSpec((512, 2048), lambda i, j: (i, j)),
        out_shape=jax.ShapeDtypeStruct((N, N), jnp.float32),
    )(zp, zp)

    return zp[:N], adj


# speedup vs baseline: 1.1939x; 1.0375x over previous
import functools

import jax
import jax.numpy as jnp
from jax import lax
from jax.experimental import pallas as pl
from jax.experimental.pallas import tpu as pltpu
from jax.experimental.pallas import tpu_sc as plsc

N = 10000
E = 320000
D_IN = 512
H1 = 800
H2 = 256

NPAD = 10240
TRASH = N
F1 = 64
F2 = 64

DEG_ACC = 20480
DEG_TRASH = 2 * N
DEG_BLOCKS = 157
DEG_PER_TILE = DEG_BLOCKS * 128
DEG_W = 16

SC_BLOCKS = 158
SC_PER_TILE = SC_BLOCKS * 128

@functools.cache
def _make_deg_kernel():
    mesh = plsc.VectorSubcoreMesh(core_axis_name="c", subcore_axis_name="s")
    return functools.partial(
        pl.kernel,
        out_type=jax.ShapeDtypeStruct((2, DEG_ACC, DEG_W), jnp.float32),
        mesh=mesh,
        scratch_types=[
            pltpu.VMEM((DEG_BLOCKS, 128), jnp.int32),
            pltpu.VMEM((128, DEG_W), jnp.float32),
            pltpu.VMEM((128, DEG_W), jnp.float32),
            pltpu.VMEM_SHARED((DEG_ACC, DEG_W), jnp.float32),
        ],
        compiler_params=pltpu.CompilerParams(use_tc_tiling_on_sc=False),
    )(_deg_body)


def _deg_body(idx_hbm, out_hbm, idx_v, ones_v, zb_v, acc):
    c = lax.axis_index("c")
    s = lax.axis_index("s")
    w = c * 16 + s

    one = jnp.ones((DEG_W,), jnp.float32)
    zero = jnp.zeros((DEG_W,), jnp.float32)

    def fill(i, _):
        ones_v[i, :] = one
        zb_v[i, :] = zero
        return 0

    lax.fori_loop(0, 128, fill, 0)

    for p in range(10):
        pltpu.sync_copy(zb_v, acc.at[pl.ds(s * 1280 + p * 128, 128)])
    plsc.subcore_barrier()

    pltpu.sync_copy(idx_hbm.at[w], idx_v)

    def body(j, _):
        pltpu.sync_copy(ones_v, acc.at[idx_v.at[j]], add=True)
        return 0

    lax.fori_loop(0, DEG_BLOCKS, body, 0)
    plsc.subcore_barrier()

    for p in range(10):
        pltpu.sync_copy(acc.at[pl.ds(s * 1280 + p * 128, 128)], zb_v)
        pltpu.sync_copy(zb_v, out_hbm.at[c, pl.ds(s * 1280 + p * 128, 128)])


@functools.cache
def _make_scatter_kernel(f):
    nchunk = D_IN // f
    mesh = plsc.VectorSubcoreMesh(core_axis_name="c", subcore_axis_name="s")
    return functools.partial(
        pl.kernel,
        out_type=jax.ShapeDtypeStruct((NPAD, nchunk, f), jnp.float32),
        mesh=mesh,
        scratch_types=[
            pltpu.VMEM((SC_BLOCKS, 128), jnp.int32),
            pltpu.VMEM((SC_BLOCKS, 128), jnp.int32),
            pltpu.VMEM((128, f), jnp.float32),
            pltpu.VMEM((128, f), jnp.float32),
            pltpu.VMEM((64, f), jnp.float32),
            pltpu.VMEM_SHARED((NPAD, f), jnp.float32),
            pltpu.SemaphoreType.DMA,
            pltpu.SemaphoreType.DMA,
        ],
        compiler_params=pltpu.CompilerParams(use_tc_tiling_on_sc=False),
    )(functools.partial(_scatter_body, f))


def _scatter_body(f, table_hbm, srcs4_hbm, dsts_hbm, out_hbm,
                  src_v, dst_v, buf_a, buf_b, zb_v, acc, sem_a, sem_b):
    nchunk = D_IN // f
    cpc = nchunk // 2
    c = lax.axis_index("c")
    s = lax.axis_index("s")

    zero = jnp.zeros((16,), jnp.float32)

    def fill(i, _):
        for k in range(f // 16):
            zb_v[i, pl.ds(k * 16, 16)] = zero
        return 0

    lax.fori_loop(0, 64, fill, 0)

    pltpu.sync_copy(dsts_hbm.at[s], dst_v)

    for lc in range(cpc):
        chunk = cpc * c + lc
        for p in range(10):
            pltpu.sync_copy(zb_v, acc.at[pl.ds(s * 640 + p * 64, 64)])
        pltpu.sync_copy(srcs4_hbm.at[chunk * 16 + s], src_v)
        plsc.subcore_barrier()

        pltpu.async_copy(table_hbm.at[src_v.at[0]], buf_a, sem_a)
        pltpu.async_copy(table_hbm.at[src_v.at[1]], buf_b, sem_b)

        def body(i, _):
            j0 = 2 * i
            pltpu.make_async_copy(
                table_hbm.at[src_v.at[j0]], buf_a, sem_a).wait()
            pltpu.sync_copy(buf_a, acc.at[dst_v.at[j0]], add=True)

            @pl.when(j0 + 2 < SC_BLOCKS)
            def _():
                pltpu.async_copy(table_hbm.at[src_v.at[j0 + 2]], buf_a, sem_a)

            pltpu.make_async_copy(
                table_hbm.at[src_v.at[j0 + 1]], buf_b, sem_b).wait()
            pltpu.sync_copy(buf_b, acc.at[dst_v.at[j0 + 1]], add=True)

            @pl.when(j0 + 3 < SC_BLOCKS)
            def _():
                pltpu.async_copy(table_hbm.at[src_v.at[j0 + 3]], buf_b, sem_b)

            return 0

        lax.fori_loop(0, SC_BLOCKS // 2, body, 0)
        plsc.subcore_barrier()

        for p in range(5):
            pltpu.sync_copy(acc.at[pl.ds(s * 640 + p * 128, 128)], buf_a)
            pltpu.sync_copy(
                buf_a, out_hbm.at[pl.ds(s * 640 + p * 128, 128), chunk])
        plsc.subcore_barrier()


_HI = lax.Precision.HIGHEST


def _prep_body(f_ref, dout_ref, o_ref):
    scale = lax.rsqrt(jnp.maximum(dout_ref[...], 1.0))
    o_ref[...] = f_ref[...] * scale


def _mid_body(s1_ref, din_ref, dout_ref, w1_ref, b1_ref, w23_ref, o_ref):
    din = lax.rsqrt(jnp.maximum(din_ref[...], 1.0))
    dout = lax.rsqrt(jnp.maximum(dout_ref[...], 1.0))
    a = s1_ref[...] * din
    h = jnp.maximum(jnp.dot(a, w1_ref[...], precision=_HI) + b1_ref[...], 0.0)
    o_ref[...] = jnp.dot(h * dout, w23_ref[...], precision=_HI)


def _z_body(s2_ref, din_ref, noise_ref, b2_ref, b3_ref, zo_ref):
    din = lax.rsqrt(jnp.maximum(din_ref[...], 1.0))
    s2n = s2_ref[...] * din
    mean = s2n[:, :H2] + b2_ref[...]
    logs = s2n[:, H2:] + b3_ref[...]
    zo_ref[...] = mean + noise_ref[...] * jnp.exp(logs)


def _dec_body(zi_ref, zj_ref, o_ref):
    a = lax.dot_general(zi_ref[...], zj_ref[...], (((1,), (1,)), ((), ())),
                        precision=lax.Precision.DEFAULT,
                        preferred_element_type=jnp.float32)
    o_ref[...] = jax.nn.sigmoid(a)


@jax.jit
def kernel(features, edge_index, noise, W1, b1, W2, b2, W3, b3):
    src = edge_index[0]
    dst = edge_index[1]

    cat = jnp.concatenate([src, dst + N])
    npad_d = 32 * DEG_PER_TILE - 2 * E
    cat = jnp.concatenate(
        [cat, DEG_TRASH + jnp.arange(npad_d, dtype=jnp.int32)
              % (DEG_ACC - DEG_TRASH)])
    deg_idx = cat.reshape(32, DEG_BLOCKS, 128)

    degp = _make_deg_kernel()(deg_idx)
    dsum = degp[0, :, 0] + degp[1, :, 0]
    deg_out = jnp.pad(dsum[:N], (0, NPAD - N), constant_values=1.0)
    deg_out = deg_out.reshape(NPAD, 1)
    deg_in = jnp.pad(dsum[N:2 * N], (0, NPAD - N), constant_values=1.0)
    deg_in = deg_in.reshape(NPAD, 1)

    src_pad = jnp.concatenate(
        [src, jnp.zeros((16 * SC_PER_TILE - E,), jnp.int32)])
    npad_e = 16 * SC_PER_TILE - E
    trash_rows = TRASH + jnp.arange(npad_e, dtype=jnp.int32) % (NPAD - N)
    dst_pad = jnp.concatenate([dst, trash_rows])
    src_slab = src_pad.reshape(16, SC_BLOCKS, 128)

    def chunked_srcs(nchunk):
        a = (src_slab[None] * nchunk +
             jnp.arange(nchunk, dtype=jnp.int32)[:, None, None, None])
        return a.reshape(nchunk * 16, SC_BLOCKS, 128)

    srcs4 = chunked_srcs(D_IN // F1)
    srcs8 = chunked_srcs(D_IN // F2)
    dsts = dst_pad.reshape(16, SC_BLOCKS, 128)

    feats_pad = jnp.pad(features, ((0, NPAD - N), (0, 0)))
    xn = pl.pallas_call(
        _prep_body,
        grid=(NPAD // 512,),
        in_specs=[
            pl.BlockSpec((512, D_IN), lambda i: (i, 0)),
            pl.BlockSpec((512, 1), lambda i: (i, 0)),
        ],
        out_specs=pl.BlockSpec((512, D_IN), lambda i: (i, 0)),
        out_shape=jax.ShapeDtypeStruct((NPAD, D_IN), jnp.float32),
    )(feats_pad, deg_out)

    s1 = _make_scatter_kernel(F1)(
        xn.reshape((D_IN // F1) * NPAD, F1), srcs4, dsts)
    s1 = s1.reshape(NPAD, D_IN)

    w23 = jnp.concatenate([W2, W3], axis=1)
    t = pl.pallas_call(
        _mid_body,
        grid=(NPAD // 512,),
        in_specs=[
            pl.BlockSpec((512, D_IN), lambda i: (i, 0)),
            pl.BlockSpec((512, 1), lambda i: (i, 0)),
            pl.BlockSpec((512, 1), lambda i: (i, 0)),
            pl.BlockSpec((D_IN, H1), lambda i: (0, 0)),
            pl.BlockSpec((1, H1), lambda i: (0, 0)),
            pl.BlockSpec((H1, 2 * H2), lambda i: (0, 0)),
        ],
        out_specs=pl.BlockSpec((512, D_IN), lambda i: (i, 0)),
        out_shape=jax.ShapeDtypeStruct((NPAD, D_IN), jnp.float32),
    )(s1, deg_in, deg_out, W1, b1.reshape(1, H1), w23)

    s2 = _make_scatter_kernel(F2)(
        t.reshape((D_IN // F2) * NPAD, F2), srcs8, dsts)
    s2 = s2.reshape(NPAD, D_IN)

    noise_pad = jnp.pad(noise, ((0, NPAD - N), (0, 0)))
    zp = pl.pallas_call(
        _z_body,
        grid=(NPAD // 512,),
        in_specs=[
            pl.BlockSpec((512, D_IN), lambda i: (i, 0)),
            pl.BlockSpec((512, 1), lambda i: (i, 0)),
            pl.BlockSpec((512, H2), lambda i: (i, 0)),
            pl.BlockSpec((1, H2), lambda i: (0, 0)),
            pl.BlockSpec((1, H2), lambda i: (0, 0)),
        ],
        out_specs=pl.BlockSpec((512, H2), lambda i: (i, 0)),
        out_shape=jax.ShapeDtypeStruct((NPAD, H2), jnp.float32),
    )(s2, deg_in, noise_pad, b2.reshape(1, H2), b3.reshape(1, H2))

    adj = pl.pallas_call(
        _dec_body,
        grid=(pl.cdiv(N, 1024), pl.cdiv(N, 2048)),
        in_specs=[
            pl.BlockSpec((1024, H2), lambda i, j: (i, 0)),
            pl.BlockSpec((2048, H2), lambda i, j: (j, 0)),
        ],
        out_specs=pl.BlockSpec((1024, 2048), lambda i, j: (i, j)),
        out_shape=jax.ShapeDtypeStruct((N, N), jnp.float32),
    )(zp, zp)

    return zp[:N], adj
